# Initial kernel scaffold; baseline (speedup 1.0000x reference)
import jax
import jax.numpy as jnp
from jax.experimental import pallas as pl


def kernel(x):
    raise NotImplementedError("write your pallas kernel here")


# speedup vs baseline: 2.3909x; 2.3909x over previous
import functools

import jax
import jax.numpy as jnp
from jax import lax
from jax.experimental import pallas as pl
from jax.experimental.pallas import tpu as pltpu
from jax.experimental.pallas import tpu_sc as plsc

R, C = 16384, 128
NC, NS, L = 2, 16, 16
NW = NC * NS
RPW = R // NW
CG = C // L
BLK = 64

_mesh = plsc.VectorSubcoreMesh(core_axis_name="c", subcore_axis_name="s")


@functools.partial(
    pl.kernel,
    mesh=_mesh,
    out_type=jax.ShapeDtypeStruct((NW, C), jnp.float32),
    scratch_types=[
        pltpu.VMEM((RPW, C), jnp.float32),
        pltpu.VMEM((1, C), jnp.float32),
    ],
)
def _partial_sums(x_hbm, out_hbm, xbuf, accbuf):
    wid = lax.axis_index("s") * NC + lax.axis_index("c")
    base = wid * RPW
    pltpu.sync_copy(x_hbm.at[pl.ds(base, RPW)], xbuf)

    def body(r, accs):
        return tuple(accs[g] + xbuf[r, pl.ds(g * L, L)] for g in range(CG))

    zeros = tuple(jnp.zeros((L,), jnp.float32) for _ in range(CG))
    accs = lax.fori_loop(0, RPW, body, zeros)
    for g in range(CG):
        accbuf[0, pl.ds(g * L, L)] = accs[g]
    pltpu.sync_copy(accbuf, out_hbm.at[pl.ds(wid, 1)])


@functools.partial(
    pl.kernel,
    mesh=_mesh,
    out_type=jax.ShapeDtypeStruct((R, C), jnp.float32),
    scratch_types=[
        pltpu.VMEM((NW, C), jnp.float32),
        pltpu.VMEM((BLK, C), jnp.float32),
    ],
)
def _broadcast_choice(ps_hbm, out_hbm, psbuf, obuf):
    wid = lax.axis_index("s") * NC + lax.axis_index("c")
    base = wid * RPW
    pltpu.sync_copy(ps_hbm, psbuf)

    big = jnp.float32(C)
    cand_v = jnp.full((L,), big, jnp.float32)
    for g in range(CG):
        def body(r, acc, g=g):
            return acc + psbuf[r, pl.ds(g * L, L)]

        s_g = lax.fori_loop(0, NW, body, jnp.zeros((L,), jnp.float32))
        lanes_f = (lax.iota(jnp.int32, L) + jnp.int32(g * L)).astype(jnp.float32)
        cand_v = jnp.minimum(cand_v, jnp.where(s_g >= 0.0, lanes_f, big))
    cand = cand_v[0]
    for j in range(1, L):
        cand = jnp.minimum(cand, cand_v[j])
    idx = jnp.where(cand >= big, jnp.int32(0), cand.astype(jnp.int32))

    for g in range(CG):
        lanes = lax.iota(jnp.int32, L) + jnp.int32(g * L)
        v_g = jnp.where(lanes == idx, 1.0, -1.0).astype(jnp.float32)

        def fill(r, _, g=g, v_g=v_g):
            obuf[r, pl.ds(g * L, L)] = v_g
            return 0

        lax.fori_loop(0, BLK, fill, 0)

    for k in range(RPW // BLK):
        pltpu.sync_copy(obuf, out_hbm.at[pl.ds(base + k * BLK, BLK)])


def kernel(x):
    return _broadcast_choice(_partial_sums(x))


# speedup vs baseline: 2.4664x; 1.0316x over previous
import functools

import jax
import jax.numpy as jnp
from jax import lax
from jax.experimental import pallas as pl
from jax.experimental.pallas import tpu as pltpu
from jax.experimental.pallas import tpu_sc as plsc

R, C = 16384, 128
NC, NS, L = 2, 16, 16
NW = NC * NS
RPW = R // NW
CG = C // L
CH = 128
NCH = RPW // CH
BLK = 128
NCP = RPW // BLK

_mesh = plsc.VectorSubcoreMesh(core_axis_name="c", subcore_axis_name="s")


@functools.partial(
    pl.kernel,
    mesh=_mesh,
    out_type=jax.ShapeDtypeStruct((NW, C), jnp.float32),
    scratch_types=[
        pltpu.VMEM((CH, C), jnp.float32),
        pltpu.VMEM((CH, C), jnp.float32),
        pltpu.VMEM((1, C), jnp.float32),
        pltpu.SemaphoreType.DMA,
        pltpu.SemaphoreType.DMA,
    ],
)
def _partial_sums(x_hbm, out_hbm, xb0, xb1, accbuf, sem0, sem1):
    wid = lax.axis_index("s") * NC + lax.axis_index("c")
    base = wid * RPW
    bufs = (xb0, xb1)
    sems = (sem0, sem1)

    def fetch(ch):
        b = ch % 2
        return pltpu.async_copy(
            x_hbm.at[pl.ds(base + ch * CH, CH)], bufs[b], sems[b])

    cps = [fetch(0), fetch(1)]
    accs = tuple(jnp.zeros((L,), jnp.float32) for _ in range(CG))
    for ch in range(NCH):
        cps[ch].wait()
        buf = bufs[ch % 2]

        def body(r, accs, buf=buf):
            return tuple(accs[g] + buf[r, pl.ds(g * L, L)] for g in range(CG))

        accs = lax.fori_loop(0, CH, body, accs, unroll=2)
        if ch + 2 < NCH:
            cps.append(fetch(ch + 2))
    for g in range(CG):
        accbuf[0, pl.ds(g * L, L)] = accs[g]
    pltpu.sync_copy(accbuf, out_hbm.at[pl.ds(wid, 1)])


@functools.partial(
    pl.kernel,
    mesh=_mesh,
    out_type=jax.ShapeDtypeStruct((R, C), jnp.float32),
    scratch_types=[
        pltpu.VMEM((NW, C), jnp.float32),
        pltpu.VMEM((BLK, C), jnp.float32),
        pltpu.SemaphoreType.DMA,
        pltpu.SemaphoreType.DMA,
    ],
)
def _broadcast_choice(ps_hbm, out_hbm, psbuf, obuf, psem, osem):
    wid = lax.axis_index("s") * NC + lax.axis_index("c")
    base = wid * RPW
    ps_cp = pltpu.async_copy(ps_hbm, psbuf, psem)

    neg = jnp.full((L,), -1.0, jnp.float32)

    def fill(r, _):
        for g in range(CG):
            obuf[r, pl.ds(g * L, L)] = neg
        return 0

    lax.fori_loop(0, BLK, fill, 0, unroll=2)
    ps_cp.wait()

    big = jnp.float32(C)
    cand_v = jnp.full((L,), big, jnp.float32)
    for g in range(CG):
        def body(r, acc, g=g):
            return acc + psbuf[r, pl.ds(g * L, L)]

        s_g = lax.fori_loop(0, NW, body, jnp.zeros((L,), jnp.float32), unroll=4)
        lanes_f = (lax.iota(jnp.int32, L) + jnp.int32(g * L)).astype(jnp.float32)
        cand_v = jnp.minimum(cand_v, jnp.where(s_g >= 0.0, lanes_f, big))
    cand = cand_v[0]
    for j in range(1, L):
        cand = jnp.minimum(cand, cand_v[j])
    idx = jnp.where(cand >= big, jnp.int32(0), cand.astype(jnp.int32))

    goff = (idx // L) * L
    lanes = lax.iota(jnp.int32, L) + goff
    v = jnp.where(lanes == idx, 1.0, -1.0).astype(jnp.float32)

    def fix(r, _):
        obuf[r, pl.ds(goff, L)] = v
        return 0

    lax.fori_loop(0, BLK, fix, 0, unroll=2)

    cps = [
        pltpu.async_copy(obuf, out_hbm.at[pl.ds(base + k * BLK, BLK)], osem)
        for k in range(NCP)
    ]
    for cp in cps:
        cp.wait()


def kernel(x):
    return _broadcast_choice(_partial_sums(x))


# speedup vs baseline: 2.7332x; 1.1082x over previous
import functools

import jax
import jax.numpy as jnp
from jax import lax
from jax.experimental import pallas as pl
from jax.experimental.pallas import tpu as pltpu
from jax.experimental.pallas import tpu_sc as plsc

R, C = 16384, 128
NC, NS, L = 2, 16, 16
NW = NC * NS
RPW = R // NW
CG = C // L
CH = 128
NCH = RPW // CH
BLK = 128
NCP = RPW // BLK

_mesh = plsc.VectorSubcoreMesh(core_axis_name="c", subcore_axis_name="s")


@functools.partial(
    pl.kernel,
    mesh=_mesh,
    out_type=jax.ShapeDtypeStruct((NW, C), jnp.float32),
    scratch_types=[
        pltpu.VMEM((CH, C), jnp.float32),
        pltpu.VMEM((CH, C), jnp.float32),
        pltpu.VMEM((1, C), jnp.float32),
        pltpu.SemaphoreType.DMA,
        pltpu.SemaphoreType.DMA,
    ],
)
def _partial_sums(x_hbm, out_hbm, xb0, xb1, accbuf, sem0, sem1):
    wid = lax.axis_index("s") * NC + lax.axis_index("c")
    base = wid * RPW
    bufs = (xb0, xb1)
    sems = (sem0, sem1)

    def fetch(ch):
        b = ch % 2
        return pltpu.async_copy(
            x_hbm.at[pl.ds(base + ch * CH, CH)], bufs[b], sems[b])

    cps = [fetch(0), fetch(1)]
    accs = tuple(jnp.zeros((L,), jnp.float32) for _ in range(CG))
    for ch in range(NCH):
        cps[ch].wait()
        buf = bufs[ch % 2]

        def body(r, accs, buf=buf):
            return tuple(accs[g] + buf[r, pl.ds(g * L, L)] for g in range(CG))

        accs = lax.fori_loop(0, CH, body, accs, unroll=2)
        if ch + 2 < NCH:
            cps.append(fetch(ch + 2))
    for g in range(CG):
        accbuf[0, pl.ds(g * L, L)] = accs[g]
    pltpu.sync_copy(accbuf, out_hbm.at[pl.ds(wid, 1)])


BR = 1024


def _tc_choice_body(ps_ref, o_ref, v_ref):
    @pl.when(pl.program_id(0) == 0)
    def _():
        s = jnp.sum(ps_ref[...], axis=0, keepdims=True)
        iota = lax.broadcasted_iota(jnp.int32, (1, C), 1)
        cand = jnp.where(s >= 0.0, iota, jnp.int32(C))
        idx = jnp.min(cand)
        idx = jnp.where(idx >= C, jnp.int32(0), idx)
        v_ref[...] = jnp.where(iota == idx, 1.0, -1.0).astype(jnp.float32)

    o_ref[...] = jnp.broadcast_to(v_ref[...], (BR, C))


def _tc_broadcast_choice(ps):
    return pl.pallas_call(
        _tc_choice_body,
        grid=(R // BR,),
        in_specs=[pl.BlockSpec((NW, C), lambda i: (0, 0))],
        out_specs=pl.BlockSpec((BR, C), lambda i: (i, 0)),
        out_shape=jax.ShapeDtypeStruct((R, C), jnp.float32),
        scratch_shapes=[pltpu.VMEM((1, C), jnp.float32)],
    )(ps)


@functools.partial(
    pl.kernel,
    mesh=_mesh,
    out_type=jax.ShapeDtypeStruct((R, C), jnp.float32),
    scratch_types=[
        pltpu.VMEM((NW, C), jnp.float32),
        pltpu.VMEM((BLK, C), jnp.float32),
        pltpu.SemaphoreType.DMA,
        pltpu.SemaphoreType.DMA,
    ],
)
def _broadcast_choice(ps_hbm, out_hbm, psbuf, obuf, psem, osem):
    wid = lax.axis_index("s") * NC + lax.axis_index("c")
    base = wid * RPW
    ps_cp = pltpu.async_copy(ps_hbm, psbuf, psem)

    neg = jnp.full((L,), -1.0, jnp.float32)

    def fill(r, _):
        for g in range(CG):
            obuf[r, pl.ds(g * L, L)] = neg
        return 0

    lax.fori_loop(0, BLK, fill, 0, unroll=2)
    ps_cp.wait()

    big = jnp.float32(C)
    cand_v = jnp.full((L,), big, jnp.float32)
    for g in range(CG):
        def body(r, acc, g=g):
            return acc + psbuf[r, pl.ds(g * L, L)]

        s_g = lax.fori_loop(0, NW, body, jnp.zeros((L,), jnp.float32), unroll=4)
        lanes_f = (lax.iota(jnp.int32, L) + jnp.int32(g * L)).astype(jnp.float32)
        cand_v = jnp.minimum(cand_v, jnp.where(s_g >= 0.0, lanes_f, big))
    cand = cand_v[0]
    for j in range(1, L):
        cand = jnp.minimum(cand, cand_v[j])
    idx = jnp.where(cand >= big, jnp.int32(0), cand.astype(jnp.int32))

    goff = (idx // L) * L
    lanes = lax.iota(jnp.int32, L) + goff
    v = jnp.where(lanes == idx, 1.0, -1.0).astype(jnp.float32)

    def fix(r, _):
        obuf[r, pl.ds(goff, L)] = v
        return 0

    lax.fori_loop(0, BLK, fix, 0, unroll=2)

    cps = [
        pltpu.async_copy(obuf, out_hbm.at[pl.ds(base + k * BLK, BLK)], osem)
        for k in range(NCP)
    ]
    for cp in cps:
        cp.wait()


def kernel(x):
    return _tc_broadcast_choice(_partial_sums(x))


# speedup vs baseline: 3.1773x; 1.1625x over previous
import functools

import jax
import jax.numpy as jnp
from jax import lax
from jax.experimental import pallas as pl
from jax.experimental.pallas import tpu as pltpu
from jax.experimental.pallas import tpu_sc as plsc

R, C = 16384, 128
NC, NS, L = 2, 16, 16
NW = NC * NS
HALF = R // 2
RPW = HALF // NW
CG = C // L
CH = 128
NCH = RPW // CH
RB = 1024
BR = 2048

_mesh = plsc.VectorSubcoreMesh(core_axis_name="c", subcore_axis_name="s")


@functools.partial(
    pl.kernel,
    mesh=_mesh,
    out_type=jax.ShapeDtypeStruct((NW, C), jnp.float32),
    scratch_types=[
        pltpu.VMEM((CH, C), jnp.float32),
        pltpu.VMEM((CH, C), jnp.float32),
        pltpu.VMEM((1, C), jnp.float32),
        pltpu.SemaphoreType.DMA,
        pltpu.SemaphoreType.DMA,
    ],
)
def _sc_partial_sums(x_hbm, out_hbm, xb0, xb1, accbuf, sem0, sem1):
    wid = lax.axis_index("s") * NC + lax.axis_index("c")
    base = wid * RPW
    bufs = (xb0, xb1)
    sems = (sem0, sem1)

    cps = [
        pltpu.async_copy(x_hbm.at[pl.ds(base + ch * CH, CH)], bufs[ch], sems[ch])
        for ch in range(NCH)
    ]
    accs = tuple(jnp.zeros((L,), jnp.float32) for _ in range(CG))
    for ch in range(NCH):
        cps[ch].wait()
        buf = bufs[ch]

        def body(r, accs, buf=buf):
            return tuple(accs[g] + buf[r, pl.ds(g * L, L)] for g in range(CG))

        accs = lax.fori_loop(0, CH, body, accs, unroll=2)
    for g in range(CG):
        accbuf[0, pl.ds(g * L, L)] = accs[g]
    pltpu.sync_copy(accbuf, out_hbm.at[pl.ds(wid, 1)])


def _tc_reduce_body(x_ref, o_ref):
    part = jnp.sum(x_ref[...], axis=0, keepdims=True)

    @pl.when(pl.program_id(0) == 0)
    def _():
        o_ref[...] = part

    @pl.when(pl.program_id(0) > 0)
    def _():
        o_ref[...] += part


def _tc_partial_sums(x):
    nhalf = HALF // RB
    return pl.pallas_call(
        _tc_reduce_body,
        grid=(nhalf,),
        in_specs=[pl.BlockSpec((RB, C), lambda i: (HALF // RB + i, 0))],
        out_specs=pl.BlockSpec((1, C), lambda i: (0, 0)),
        out_shape=jax.ShapeDtypeStruct((1, C), jnp.float32),
    )(x)


def _tc_choice_body(ps_ref, pt_ref, o_ref, v_ref):
    @pl.when(pl.program_id(0) == 0)
    def _():
        s = jnp.sum(ps_ref[...], axis=0, keepdims=True) + pt_ref[...]
        iota = lax.broadcasted_iota(jnp.int32, (1, C), 1)
        cand = jnp.where(s >= 0.0, iota, jnp.int32(C))
        idx = jnp.min(cand)
        idx = jnp.where(idx >= C, jnp.int32(0), idx)
        v_ref[...] = jnp.where(iota == idx, 1.0, -1.0).astype(jnp.float32)

    o_ref[...] = jnp.broadcast_to(v_ref[...], (BR, C))


def _tc_broadcast_choice(ps_sc, ps_tc):
    return pl.pallas_call(
        _tc_choice_body,
        grid=(R // BR,),
        in_specs=[
            pl.BlockSpec((NW, C), lambda i: (0, 0)),
            pl.BlockSpec((1, C), lambda i: (0, 0)),
        ],
        out_specs=pl.BlockSpec((BR, C), lambda i: (i, 0)),
        out_shape=jax.ShapeDtypeStruct((R, C), jnp.float32),
        scratch_shapes=[pltpu.VMEM((1, C), jnp.float32)],
    )(ps_sc, ps_tc)


def kernel(x):
    ps_sc = _sc_partial_sums(x)
    ps_tc = _tc_partial_sums(x)
    return _tc_broadcast_choice(ps_sc, ps_tc)


# speedup vs baseline: 3.2734x; 1.0302x over previous
import functools

import jax
import jax.numpy as jnp
from jax import lax
from jax.experimental import pallas as pl
from jax.experimental.pallas import tpu as pltpu
from jax.experimental.pallas import tpu_sc as plsc

R, C = 16384, 128
NC, NS, L = 2, 16, 16
NW = NC * NS
HALF = R // 2
RPW = HALF // NW
CG = C // L
CH = 128
NCH = RPW // CH
RB = 1024
BR = 4096

_mesh = plsc.VectorSubcoreMesh(core_axis_name="c", subcore_axis_name="s")


@functools.partial(
    pl.kernel,
    mesh=_mesh,
    out_type=jax.ShapeDtypeStruct((NW, C), jnp.float32),
    scratch_types=[
        pltpu.VMEM((CH, C), jnp.float32),
        pltpu.VMEM((CH, C), jnp.float32),
        pltpu.VMEM((1, C), jnp.float32),
        pltpu.SemaphoreType.DMA,
        pltpu.SemaphoreType.DMA,
    ],
)
def _sc_partial_sums(x_hbm, out_hbm, xb0, xb1, accbuf, sem0, sem1):
    wid = lax.axis_index("s") * NC + lax.axis_index("c")
    base = wid * RPW
    bufs = (xb0, xb1)
    sems = (sem0, sem1)

    cps = [
        pltpu.async_copy(x_hbm.at[pl.ds(base + ch * CH, CH)], bufs[ch], sems[ch])
        for ch in range(NCH)
    ]
    accs = tuple(jnp.zeros((L,), jnp.float32) for _ in range(CG))
    for ch in range(NCH):
        cps[ch].wait()
        buf = bufs[ch]

        def body(r, accs, buf=buf):
            return tuple(accs[g] + buf[r, pl.ds(g * L, L)] for g in range(CG))

        accs = lax.fori_loop(0, CH, body, accs, unroll=4)
    for g in range(CG):
        accbuf[0, pl.ds(g * L, L)] = accs[g]
    pltpu.sync_copy(accbuf, out_hbm.at[pl.ds(wid, 1)])


def _tc_reduce_body(x_ref, o_ref):
    part = jnp.sum(x_ref[...], axis=0, keepdims=True)

    @pl.when(pl.program_id(0) == 0)
    def _():
        o_ref[...] = part

    @pl.when(pl.program_id(0) > 0)
    def _():
        o_ref[...] += part


def _tc_partial_sums(x):
    nhalf = HALF // RB
    return pl.pallas_call(
        _tc_reduce_body,
        grid=(nhalf,),
        in_specs=[pl.BlockSpec((RB, C), lambda i: (HALF // RB + i, 0))],
        out_specs=pl.BlockSpec((1, C), lambda i: (0, 0)),
        out_shape=jax.ShapeDtypeStruct((1, C), jnp.float32),
    )(x)


def _tc_choice_body(ps_ref, pt_ref, o_ref, v_ref):
    @pl.when(pl.program_id(0) == 0)
    def _():
        s = jnp.sum(ps_ref[...], axis=0, keepdims=True) + pt_ref[...]
        iota = lax.broadcasted_iota(jnp.int32, (1, C), 1)
        cand = jnp.where(s >= 0.0, iota, jnp.int32(C))
        idx = jnp.min(cand)
        idx = jnp.where(idx >= C, jnp.int32(0), idx)
        v_ref[...] = jnp.where(iota == idx, 1.0, -1.0).astype(jnp.float32)

    o_ref[...] = jnp.broadcast_to(v_ref[...], (BR, C))


def _tc_broadcast_choice(ps_sc, ps_tc):
    return pl.pallas_call(
        _tc_choice_body,
        grid=(R // BR,),
        in_specs=[
            pl.BlockSpec((NW, C), lambda i: (0, 0)),
            pl.BlockSpec((1, C), lambda i: (0, 0)),
        ],
        out_specs=pl.BlockSpec((BR, C), lambda i: (i, 0)),
        out_shape=jax.ShapeDtypeStruct((R, C), jnp.float32),
        scratch_shapes=[pltpu.VMEM((1, C), jnp.float32)],
    )(ps_sc, ps_tc)


def kernel(x):
    ps_sc = _sc_partial_sums(x)
    ps_tc = _tc_partial_sums(x)
    return _tc_broadcast_choice(ps_sc, ps_tc)


# speedup vs baseline: 3.3236x; 1.0153x over previous
import functools

import jax
import jax.numpy as jnp
from jax import lax
from jax.experimental import pallas as pl
from jax.experimental.pallas import tpu as pltpu
from jax.experimental.pallas import tpu_sc as plsc

R, C = 16384, 128
NC, NS, L = 2, 16, 16
NW = NC * NS
SCROWS = 6144
RPW = SCROWS // NW
CG = C // L
CH = 96
NCH = RPW // CH
RB = 2048
BR = 4096

_mesh = plsc.VectorSubcoreMesh(core_axis_name="c", subcore_axis_name="s")


@functools.partial(
    pl.kernel,
    mesh=_mesh,
    out_type=jax.ShapeDtypeStruct((NW, C), jnp.float32),
    scratch_types=[
        pltpu.VMEM((CH, C), jnp.float32),
        pltpu.VMEM((CH, C), jnp.float32),
        pltpu.VMEM((1, C), jnp.float32),
        pltpu.SemaphoreType.DMA,
        pltpu.SemaphoreType.DMA,
    ],
)
def _sc_partial_sums(x_hbm, out_hbm, xb0, xb1, accbuf, sem0, sem1):
    wid = lax.axis_index("s") * NC + lax.axis_index("c")
    base = wid * RPW
    bufs = (xb0, xb1)
    sems = (sem0, sem1)

    cps = [
        pltpu.async_copy(x_hbm.at[pl.ds(base + ch * CH, CH)], bufs[ch], sems[ch])
        for ch in range(NCH)
    ]
    accs = tuple(jnp.zeros((L,), jnp.float32) for _ in range(CG))
    for ch in range(NCH):
        cps[ch].wait()
        buf = bufs[ch]

        def body(r, accs, buf=buf):
            return tuple(accs[g] + buf[r, pl.ds(g * L, L)] for g in range(CG))

        accs = lax.fori_loop(0, CH, body, accs, unroll=4)
    for g in range(CG):
        accbuf[0, pl.ds(g * L, L)] = accs[g]
    pltpu.sync_copy(accbuf, out_hbm.at[pl.ds(wid, 1)])


def _tc_reduce_body(x_ref, o_ref):
    part = jnp.sum(x_ref[...], axis=0, keepdims=True)

    @pl.when(pl.program_id(0) == 0)
    def _():
        o_ref[...] = part

    @pl.when(pl.program_id(0) > 0)
    def _():
        o_ref[...] += part


def _tc_partial_sums(x):
    nblk = (R - SCROWS) // RB
    return pl.pallas_call(
        _tc_reduce_body,
        grid=(nblk,),
        in_specs=[pl.BlockSpec((RB, C), lambda i: (SCROWS // RB + i, 0))],
        out_specs=pl.BlockSpec((1, C), lambda i: (0, 0)),
        out_shape=jax.ShapeDtypeStruct((1, C), jnp.float32),
    )(x)


def _tc_choice_body(ps_ref, pt_ref, o_ref, v_ref):
    @pl.when(pl.program_id(0) == 0)
    def _():
        s = jnp.sum(ps_ref[...], axis=0, keepdims=True) + pt_ref[...]
        iota = lax.broadcasted_iota(jnp.int32, (1, C), 1)
        cand = jnp.where(s >= 0.0, iota, jnp.int32(C))
        idx = jnp.min(cand)
        idx = jnp.where(idx >= C, jnp.int32(0), idx)
        v_ref[...] = jnp.where(iota == idx, 1.0, -1.0).astype(jnp.float32)

    o_ref[...] = jnp.broadcast_to(v_ref[...], (BR, C))


def _tc_broadcast_choice(ps_sc, ps_tc):
    return pl.pallas_call(
        _tc_choice_body,
        grid=(R // BR,),
        in_specs=[
            pl.BlockSpec((NW, C), lambda i: (0, 0)),
            pl.BlockSpec((1, C), lambda i: (0, 0)),
        ],
        out_specs=pl.BlockSpec((BR, C), lambda i: (i, 0)),
        out_shape=jax.ShapeDtypeStruct((R, C), jnp.float32),
        scratch_shapes=[pltpu.VMEM((1, C), jnp.float32)],
    )(ps_sc, ps_tc)


def kernel(x):
    ps_sc = _sc_partial_sums(x)
    ps_tc = _tc_partial_sums(x)
    return _tc_broadcast_choice(ps_sc, ps_tc)


# speedup vs baseline: 3.3830x; 1.0179x over previous
import functools

import jax
import jax.numpy as jnp
from jax import lax
from jax.experimental import pallas as pl
from jax.experimental.pallas import tpu as pltpu
from jax.experimental.pallas import tpu_sc as plsc

R, C = 16384, 128
NC, NS, L = 2, 16, 16
NW = NC * NS
SCROWS = 4096
RPW = SCROWS // NW
CG = C // L
CH = 64
NCH = RPW // CH
RB = 2048
BR = 4096

_mesh = plsc.VectorSubcoreMesh(core_axis_name="c", subcore_axis_name="s")


@functools.partial(
    pl.kernel,
    mesh=_mesh,
    out_type=jax.ShapeDtypeStruct((NW, C), jnp.float32),
    scratch_types=[
        pltpu.VMEM((CH, C), jnp.float32),
        pltpu.VMEM((CH, C), jnp.float32),
        pltpu.VMEM((1, C), jnp.float32),
        pltpu.SemaphoreType.DMA,
        pltpu.SemaphoreType.DMA,
    ],
)
def _sc_partial_sums(x_hbm, out_hbm, xb0, xb1, accbuf, sem0, sem1):
    wid = lax.axis_index("s") * NC + lax.axis_index("c")
    base = wid * RPW
    bufs = (xb0, xb1)
    sems = (sem0, sem1)

    cps = [
        pltpu.async_copy(x_hbm.at[pl.ds(base + ch * CH, CH)], bufs[ch], sems[ch])
        for ch in range(NCH)
    ]
    accs = tuple(jnp.zeros((L,), jnp.float32) for _ in range(CG))
    for ch in range(NCH):
        cps[ch].wait()
        buf = bufs[ch]

        def body(r, accs, buf=buf):
            return tuple(accs[g] + buf[r, pl.ds(g * L, L)] for g in range(CG))

        accs = lax.fori_loop(0, CH, body, accs, unroll=4)
    for g in range(CG):
        accbuf[0, pl.ds(g * L, L)] = accs[g]
    pltpu.sync_copy(accbuf, out_hbm.at[pl.ds(wid, 1)])


def _tc_reduce_body(x_ref, o_ref):
    part = jnp.sum(x_ref[...], axis=0, keepdims=True)

    @pl.when(pl.program_id(0) == 0)
    def _():
        o_ref[...] = part

    @pl.when(pl.program_id(0) > 0)
    def _():
        o_ref[...] += part


def _tc_partial_sums(x):
    nblk = (R - SCROWS) // RB
    return pl.pallas_call(
        _tc_reduce_body,
        grid=(nblk,),
        in_specs=[pl.BlockSpec((RB, C), lambda i: (SCROWS // RB + i, 0))],
        out_specs=pl.BlockSpec((1, C), lambda i: (0, 0)),
        out_shape=jax.ShapeDtypeStruct((1, C), jnp.float32),
    )(x)


def _tc_choice_body(ps_ref, pt_ref, o_ref, v_ref):
    @pl.when(pl.program_id(0) == 0)
    def _():
        s = jnp.sum(ps_ref[...], axis=0, keepdims=True) + pt_ref[...]
        iota = lax.broadcasted_iota(jnp.int32, (1, C), 1)
        cand = jnp.where(s >= 0.0, iota, jnp.int32(C))
        idx = jnp.min(cand)
        idx = jnp.where(idx >= C, jnp.int32(0), idx)
        v_ref[...] = jnp.where(iota == idx, 1.0, -1.0).astype(jnp.float32)

    o_ref[...] = jnp.broadcast_to(v_ref[...], (BR, C))


def _tc_broadcast_choice(ps_sc, ps_tc):
    return pl.pallas_call(
        _tc_choice_body,
        grid=(R // BR,),
        in_specs=[
            pl.BlockSpec((NW, C), lambda i: (0, 0)),
            pl.BlockSpec((1, C), lambda i: (0, 0)),
        ],
        out_specs=pl.BlockSpec((BR, C), lambda i: (i, 0)),
        out_shape=jax.ShapeDtypeStruct((R, C), jnp.float32),
        scratch_shapes=[pltpu.VMEM((1, C), jnp.float32)],
    )(ps_sc, ps_tc)


def kernel(x):
    ps_sc = _sc_partial_sums(x)
    ps_tc = _tc_partial_sums(x)
    return _tc_broadcast_choice(ps_sc, ps_tc)
